# Initial kernel scaffold; baseline (speedup 1.0000x reference)
import jax
import jax.numpy as jnp
from jax.experimental import pallas as pl


def kernel(feat_0, coord_0, feat_1, coord_1):
    raise NotImplementedError("write your pallas kernel here")


# speedup vs baseline: 7.6032x; 7.6032x over previous
import functools

import jax
import jax.numpy as jnp
from jax import lax
from jax.experimental import pallas as pl
from jax.experimental.pallas import tpu as pltpu
from jax.experimental.pallas import tpu_sc as plsc

TEMP = 0.1
KNN = 8
SINK_ITER = 3
N0, N1, C = 8192, 4096, 512

NC, NS, L = 2, 16, 16
NW = NC * NS
BQ = 256
CQ = 8


def _norm_body(f_ref, o_ref):
    x = f_ref[...]
    n = jnp.sqrt(jnp.sum(x * x, axis=1, keepdims=True))
    o_ref[...] = x / jnp.maximum(n, 1e-12)


def _normalize(f):
    n, c = f.shape
    blk = 1024
    return pl.pallas_call(
        _norm_body,
        grid=(n // blk,),
        in_specs=[pl.BlockSpec((blk, c), lambda i: (i, 0))],
        out_specs=pl.BlockSpec((blk, c), lambda i: (i, 0)),
        out_shape=jax.ShapeDtypeStruct((n, c), jnp.float32),
    )(f)


def _knn_body(cq_ref, cpt_ref, idx_ref):
    cq = cq_ref[...]
    cpt = cpt_ref[...]
    sqq = jnp.sum(cq * cq, axis=1, keepdims=True)
    sqp = jnp.sum(cpt * cpt, axis=0, keepdims=True)
    mm = lax.dot_general(cq, cpt, (((1,), (0,)), ((), ())),
                         preferred_element_type=jnp.float32)
    d2 = (sqq - 2.0 * mm) + sqp
    iota = lax.broadcasted_iota(jnp.int32, d2.shape, 1)
    big_i = jnp.int32(2 ** 30)
    inf = jnp.float32(jnp.inf)
    cols = []
    for _ in range(KNN):
        m = jnp.min(d2, axis=1, keepdims=True)
        cand = jnp.where(d2 == m, iota, big_i)
        ik = jnp.min(cand, axis=1, keepdims=True)
        cols.append(ik)
        d2 = jnp.where(iota == ik, inf, d2)
    idx_ref[...] = jnp.concatenate(cols, axis=1)


def _knn(cq_pad, cpt_pad):
    nq = cq_pad.shape[0]
    m = cpt_pad.shape[1]
    return pl.pallas_call(
        _knn_body,
        grid=(nq // BQ,),
        in_specs=[
            pl.BlockSpec((BQ, 8), lambda i: (i, 0)),
            pl.BlockSpec((8, m), lambda i: (0, 0)),
        ],
        out_specs=pl.BlockSpec((BQ, KNN), lambda i: (i, 0)),
        out_shape=jax.ShapeDtypeStruct((nq, KNN), jnp.int32),
    )(cq_pad, cpt_pad)


def _sc_dot_phase(tab_hbm, idxf_hbm, q_hbm, out_hbm, nq_w, base_q,
                  idx_v, rows_v, fq_v, sim_v, sem):
    pltpu.sync_copy(idxf_hbm.at[pl.ds(base_q * KNN, nq_w * KNN)],
                    idx_v.at[pl.ds(0, nq_w * KNN)])
    nchunks = nq_w // CQ

    def chunk_body(ci, _):
        qb = ci * CQ
        pltpu.async_copy(tab_hbm.at[idx_v.at[pl.ds(qb * KNN, CQ * KNN)]],
                         rows_v, sem).wait()
        pltpu.sync_copy(q_hbm.at[pl.ds(base_q + qb, CQ)], fq_v)
        lane = lax.broadcasted_iota(jnp.int32, (L,), 0)

        def pair_body(pi, _):
            q0 = 2 * pi
            accs = [jnp.zeros((L,), jnp.float32) for _ in range(2 * KNN)]
            for c in range(C // L):
                sl = pl.ds(c * L, L)
                f0 = fq_v[q0, sl]
                f1 = fq_v[q0 + 1, sl]
                for k in range(KNN):
                    accs[k] = accs[k] + f0 * rows_v[q0 * KNN + k, sl]
                    accs[KNN + k] = accs[KNN + k] + f1 * rows_v[(q0 + 1) * KNN + k, sl]
            v = jnp.zeros((L,), jnp.float32)
            for j in range(2 * KNN):
                s = accs[j]
                for sh in (8, 4, 2, 1):
                    s = s + s.at[lane ^ sh].get(mode="promise_in_bounds")
                v = jnp.where(lane == j, s / TEMP, v)
            sim_v[pl.ds((qb + q0) * KNN, 2 * KNN)] = v
            return 0

        lax.fori_loop(0, CQ // 2, pair_body, 0)
        return 0

    lax.fori_loop(0, nchunks, chunk_body, 0)
    pltpu.sync_copy(sim_v.at[pl.ds(0, nq_w * KNN)],
                    out_hbm.at[pl.ds(base_q * KNN, nq_w * KNN)])


def _sc_sim_kernel(f0n_hbm, f1n_hbm, idxj_hbm, idxi_hbm,
                   simi_hbm, simj_hbm, idx_v, rows_v, fq_v, sim_v, sem):
    wid = lax.axis_index("c") * NS + lax.axis_index("s")
    nq0 = N0 // NW
    nq1 = N1 // NW
    _sc_dot_phase(f1n_hbm, idxj_hbm, f0n_hbm, simi_hbm, nq0, wid * nq0,
                  idx_v, rows_v, fq_v, sim_v, sem)
    _sc_dot_phase(f0n_hbm, idxi_hbm, f1n_hbm, simj_hbm, nq1, wid * nq1,
                  idx_v, rows_v, fq_v, sim_v, sem)


def _sc_sim(f0n, f1n, idxj_flat, idxi_flat):
    nq0 = N0 // NW
    mesh = plsc.VectorSubcoreMesh(core_axis_name="c", subcore_axis_name="s",
                                  num_cores=NC, num_subcores=NS)
    run = pl.kernel(
        _sc_sim_kernel,
        out_type=(jax.ShapeDtypeStruct((N0 * KNN,), jnp.float32),
                  jax.ShapeDtypeStruct((N1 * KNN,), jnp.float32)),
        mesh=mesh,
        scratch_types=[
            pltpu.VMEM((nq0 * KNN,), jnp.int32),
            pltpu.VMEM((CQ * KNN, C), jnp.float32),
            pltpu.VMEM((CQ, C), jnp.float32),
            pltpu.VMEM((nq0 * KNN,), jnp.float32),
            pltpu.SemaphoreType.DMA,
        ],
    )
    return run(f0n, f1n, idxj_flat, idxi_flat)


def _loss_body(si_ref, sj_ref, o_ref):
    def dir_loss(s):
        q = jnp.exp(s)
        q = q / jnp.sum(q)
        for _ in range(SINK_ITER):
            q = q / jnp.sum(q, axis=0, keepdims=True)
            q = q / jnp.sum(q, axis=1, keepdims=True)
        e = jnp.exp(s - jnp.max(s, axis=1, keepdims=True))
        p = e / jnp.sum(e, axis=1, keepdims=True)
        return jnp.mean(-jnp.log(jnp.sum(q * p, axis=1) + 1e-12))

    total = 0.5 * (dir_loss(si_ref[...]) + dir_loss(sj_ref[...]))
    o_ref[...] = total[None, None]


def _loss(sim_i, sim_j):
    return pl.pallas_call(
        _loss_body,
        out_shape=jax.ShapeDtypeStruct((1, 1), jnp.float32),
    )(sim_i, sim_j)


def kernel(feat_0, coord_0, feat_1, coord_1):
    c0p = jnp.pad(coord_0, ((0, 0), (0, 5)))
    c1p = jnp.pad(coord_1, ((0, 0), (0, 5)))
    f0n = _normalize(feat_0)
    f1n = _normalize(feat_1)
    idx_j = _knn(c0p, c1p.T)
    idx_i = _knn(c1p, c0p.T)
    sim_i_f, sim_j_f = _sc_sim(f0n, f1n, idx_j.reshape(-1), idx_i.reshape(-1))
    out = _loss(sim_i_f.reshape(N0, KNN), sim_j_f.reshape(N1, KNN))
    return out.reshape(())
